# baseline (speedup 1.0000x reference)
import jax
import jax.numpy as jnp
from jax.experimental import pallas as pl
from jax.experimental.pallas import tpu as pltpu

_BETA = 0.25
_B = 8192
_K = 8192
_D = 256
_BB = 512
_KB = 2048
_W0 = 2736
_W1 = 5472

_BIG = 2 ** 30


def _dist(s):
    return jnp.sqrt(s) * 0.625


def _vq_body(x_ref, c_ref, x2_ref, c2_ref, idx_ref, q_ref, loss_ref,
             sq_ref, acc_ref):
    i = pl.program_id(0)
    nb = pl.num_programs(0)
    x = x_ref[...]
    xx2 = x + x
    x2 = x2_ref[...]

    wmins = []
    for j in range(_K // _KB):
        cch = c_ref[j * _KB:(j + 1) * _KB, :]
        c2 = c2_ref[0:1, j * _KB:(j + 1) * _KB]
        mm2 = jax.lax.dot_general(
            xx2, cch, dimension_numbers=(((1,), (1,)), ((), ())),
            preferred_element_type=jnp.float32,
            precision=jax.lax.Precision.DEFAULT)
        sq = jnp.maximum((x2 - mm2) + c2, 0.0)
        sq_ref[:, j * _KB:(j + 1) * _KB] = sq
        lo, hi = j * _KB, (j + 1) * _KB
        bnd = _W0 if lo < _W0 < hi else (_W1 if lo < _W1 < hi else None)
        if bnd is None:
            wmins.append(jnp.min(sq, axis=1, keepdims=True))
        else:
            iota = jax.lax.broadcasted_iota(jnp.int32, (_BB, _KB), 1) + lo
            msk = iota < bnd
            wmins.append(jnp.min(jnp.where(msk, sq, jnp.inf), axis=1, keepdims=True))
            wmins.append(jnp.min(jnp.where(msk, jnp.inf, sq), axis=1, keepdims=True))

    m_w = jnp.concatenate([jnp.minimum(wmins[0], wmins[1]),
                           jnp.minimum(wmins[2], wmins[3]),
                           jnp.minimum(wmins[4], wmins[5])], axis=1)
    d_w = _dist(m_w)

    lo0 = jax.lax.bitcast_convert_type(m_w, jnp.int32)
    hi0 = jnp.full(lo0.shape, 0x7F800000, jnp.int32)

    def _step(_, carry):
        lo, hi = carry
        mid = lo + ((hi - lo) >> 1)
        ok = _dist(jax.lax.bitcast_convert_type(mid, jnp.float32)) <= d_w
        return jnp.where(ok, mid, lo), jnp.where(ok, hi, mid)

    lo_f, _ = jax.lax.fori_loop(0, 31, _step, (lo0, hi0))
    b_w = jax.lax.bitcast_convert_type(lo_f, jnp.float32)

    firsts = [jnp.full((_BB, 1), _BIG, jnp.int32)] * 3
    for j in range(_K // _KB):
        sq = sq_ref[:, j * _KB:(j + 1) * _KB]
        lo, hi = j * _KB, (j + 1) * _KB
        iota = jax.lax.broadcasted_iota(jnp.int32, (_BB, _KB), 1) + lo
        bnd = _W0 if lo < _W0 < hi else (_W1 if lo < _W1 < hi else None)
        w = (0 if hi <= _W0 else (1 if hi <= _W1 else 2))
        if bnd is None:
            cand = sq <= b_w[:, w:w + 1]
            a = jnp.min(jnp.where(cand, iota, _BIG), axis=1, keepdims=True)
            firsts[w] = jnp.minimum(firsts[w], a)
        else:
            msk = iota < bnd
            cl = msk & (sq <= b_w[:, w - 1:w])
            cr = (~msk) & (sq <= b_w[:, w:w + 1])
            firsts[w - 1] = jnp.minimum(
                firsts[w - 1],
                jnp.min(jnp.where(cl, iota, _BIG), axis=1, keepdims=True))
            firsts[w] = jnp.minimum(
                firsts[w],
                jnp.min(jnp.where(cr, iota, _BIG), axis=1, keepdims=True))

    run_v = jnp.full((_BB,), jnp.inf, jnp.float32)
    run_i = jnp.zeros((_BB,), jnp.int32)
    for w in range(3):
        m = d_w[:, w]
        a = firsts[w][:, 0]
        upd = m < run_v
        run_i = jnp.where(upd, a, run_i)
        run_v = jnp.where(upd, m.astype(jnp.bfloat16).astype(jnp.float32), run_v)
    idx_ref[...] = run_i
    msq = jnp.min(m_w, axis=1)

    q = jnp.zeros((_BB, _D), jnp.float32)
    for j in range(_K // _KB):
        cch = c_ref[j * _KB:(j + 1) * _KB, :]
        iota = jax.lax.broadcasted_iota(jnp.int32, (_BB, _KB), 1) + j * _KB
        onehot = jnp.where(run_i[:, None] == iota, 1.0, 0.0)
        q = q + jax.lax.dot_general(
            onehot, cch, dimension_numbers=(((1,), (0,)), ((), ())),
            preferred_element_type=jnp.float32,
            precision=jax.lax.Precision.DEFAULT)
    q_ref[...] = q

    @pl.when(i == 0)
    def _():
        acc_ref[0] = 0.0
    acc_ref[0] += jnp.sum(msq)

    @pl.when(i == nb - 1)
    def _():
        loss_ref[0, 0] = acc_ref[0] * ((1.0 + _BETA) / _B)


def kernel(x, codes):
    c = codes[0]
    x2 = jnp.sum(x * x, axis=-1, keepdims=True)
    c2 = jnp.sum(c * c, axis=-1)[None, :]
    idx, q, loss = pl.pallas_call(
        _vq_body,
        grid=(_B // _BB,),
        in_specs=[
            pl.BlockSpec((_BB, _D), lambda i: (i, 0)),
            pl.BlockSpec((_K, _D), lambda i: (0, 0)),
            pl.BlockSpec((_BB, 1), lambda i: (i, 0)),
            pl.BlockSpec((1, _K), lambda i: (0, 0)),
        ],
        out_specs=[
            pl.BlockSpec((_BB,), lambda i: (i,)),
            pl.BlockSpec((_BB, _D), lambda i: (i, 0)),
            pl.BlockSpec(memory_space=pltpu.SMEM, block_shape=(1, 1),
                         index_map=lambda i: (0, 0)),
        ],
        out_shape=[
            jax.ShapeDtypeStruct((_B,), jnp.int32),
            jax.ShapeDtypeStruct((_B, _D), jnp.float32),
            jax.ShapeDtypeStruct((1, 1), jnp.float32),
        ],
        scratch_shapes=[pltpu.VMEM((_BB, _K), jnp.float32),
                        pltpu.SMEM((1,), jnp.float32)],
    )(x, c, x2, c2)
    return q, idx, loss[0, 0]


# speedup vs baseline: 1.5143x; 1.5143x over previous
import functools

import jax
import jax.numpy as jnp
from jax.experimental import pallas as pl
from jax.experimental.pallas import tpu as pltpu
from jax.experimental.pallas import tpu_sc as plsc

_BETA = 0.25
_B = 8192
_K = 8192
_D = 256
_BB = 512
_KB = 2048
_W0 = 2736
_W1 = 5472
_GW = 128

_BIG = 2 ** 30


def _tile_minarg(dist, iota, mask=None):
    d = dist if mask is None else jnp.where(mask, dist, jnp.inf)
    m = jnp.min(d, axis=1)
    eq = d == m[:, None]
    a = jnp.min(jnp.where(eq, iota, _BIG), axis=1)
    return m, a


def _merge_f32(m1, a1, m2, a2):
    upd = m2 < m1
    return jnp.where(upd, m2, m1), jnp.where(upd, a2, a1)


def _vq_body(x_ref, c_ref, x2_ref, c2_ref, idx_ref, loss_ref, acc_ref):
    i = pl.program_id(0)
    nb = pl.num_programs(0)
    x = x_ref[...]
    xx2 = x + x
    x2 = x2_ref[...]

    tiles = []
    for j in range(_K // _KB):
        cch = c_ref[j * _KB:(j + 1) * _KB, :]
        c2 = c2_ref[0:1, j * _KB:(j + 1) * _KB]
        mm2 = jax.lax.dot_general(
            xx2, cch, dimension_numbers=(((1,), (1,)), ((), ())),
            preferred_element_type=jnp.float32,
            precision=jax.lax.Precision.DEFAULT)
        sq = jnp.maximum((x2 - mm2) + c2, 0.0)
        dist = jnp.sqrt(sq) * 0.625
        iota = jax.lax.broadcasted_iota(jnp.int32, (_BB, _KB), 1) + j * _KB
        lo, hi = j * _KB, (j + 1) * _KB
        bnd = _W0 if lo < _W0 < hi else (_W1 if lo < _W1 < hi else None)
        if bnd is None:
            tiles.append(_tile_minarg(dist, iota))
        else:
            tiles.append(_tile_minarg(dist, iota, mask=iota < bnd))
            tiles.append(_tile_minarg(dist, iota, mask=iota >= bnd))

    w0 = _merge_f32(*tiles[0], *tiles[1])
    w1 = _merge_f32(*tiles[2], *tiles[3])
    w2 = _merge_f32(*tiles[4], *tiles[5])

    run_v = jnp.full((_BB,), jnp.inf, jnp.float32)
    run_i = jnp.zeros((_BB,), jnp.int32)
    for m, a in (w0, w1, w2):
        upd = m < run_v
        run_i = jnp.where(upd, a, run_i)
        run_v = jnp.where(upd, m.astype(jnp.bfloat16).astype(jnp.float32), run_v)
    idx_ref[...] = run_i

    dmin = jnp.minimum(jnp.minimum(w0[0], w1[0]), w2[0])
    msq = (dmin * 1.6) ** 2
    @pl.when(i == 0)
    def _():
        acc_ref[0] = 0.0
    acc_ref[0] += jnp.sum(msq)

    @pl.when(i == nb - 1)
    def _():
        loss_ref[0, 0] = acc_ref[0] * ((1.0 + _BETA) / _B)


def _sc_gather(c, idx):
    idx2 = idx.reshape(1, _B)
    mesh = plsc.VectorSubcoreMesh(core_axis_name="c", subcore_axis_name="s")

    @functools.partial(
        pl.kernel,
        out_type=jax.ShapeDtypeStruct((_B, _D), jnp.float32),
        mesh=mesh)
    def gather_kernel(c_hbm, i_hbm, o_hbm):
        def body(i_vmem, o_vmem):
            pltpu.sync_copy(c_hbm.at[i_vmem.at[0]], o_vmem)

        pltpu.emit_pipeline(
            body,
            grid=(_B // _GW,),
            in_specs=[pl.BlockSpec((1, _GW), lambda i: (0, i))],
            out_specs=[pl.BlockSpec((_GW, _D), lambda i: (i, 0))],
            core_axis_name=("c", "s"),
            dimension_semantics=(pltpu.PARALLEL,),
        )(i_hbm, o_hbm)

    return gather_kernel(c, idx2)


def kernel(x, codes):
    c = codes[0]
    x2 = jnp.sum(x * x, axis=-1, keepdims=True)
    c2 = jnp.sum(c * c, axis=-1)[None, :]
    idx, loss = pl.pallas_call(
        _vq_body,
        grid=(_B // _BB,),
        in_specs=[
            pl.BlockSpec((_BB, _D), lambda i: (i, 0)),
            pl.BlockSpec((_K, _D), lambda i: (0, 0)),
            pl.BlockSpec((_BB, 1), lambda i: (i, 0)),
            pl.BlockSpec((1, _K), lambda i: (0, 0)),
        ],
        out_specs=[
            pl.BlockSpec((_BB,), lambda i: (i,)),
            pl.BlockSpec(memory_space=pltpu.SMEM, block_shape=(1, 1),
                         index_map=lambda i: (0, 0)),
        ],
        out_shape=[
            jax.ShapeDtypeStruct((_B,), jnp.int32),
            jax.ShapeDtypeStruct((1, 1), jnp.float32),
        ],
        scratch_shapes=[pltpu.SMEM((1,), jnp.float32)],
    )(x, c, x2, c2)
    q = _sc_gather(c, idx)
    return q, idx, loss[0, 0]
